# baseline (speedup 1.0000x reference)
import functools

import jax
import jax.numpy as jnp
import numpy as np
from jax.experimental import pallas as pl
from jax.experimental.pallas import tpu as pltpu

_FI, _FJ, _XI, _OI, _I, _J, _RS, _MV, _SLOT, _STORE, _DOS, _FIN = range(12)


def _build_schedule(B: int, K: int) -> np.ndarray:
    upper = [(i, j) for i in range(B) for j in range(i + 1, B)]
    K = min(K, len(upper))
    cached = upper[len(upper) - K:] if K else []
    slot_of = {t: s for s, t in enumerate(cached)}
    rows = []

    def step(fi, fj, xi, oi, i, j, rs, mv, slot, store, dos, fin):
        rows.append([fi, fj, xi, oi, i, j, rs, mv, slot, store, dos, fin])

    for i in range(B):
        order = list(range(i)) + list(range(i + 1, B)) + [i]
        for j in order:
            diag = j == i
            slot = slot_of.get((i, j), -1)
            step(i, j, i, B - 1, i, j, 1,
                 0 if (j < i) else -1, slot, 1 if slot >= 0 else 0,
                 1 if diag else 0,
                 B - 1 if (i == B - 1 and diag) else -1)
    prev = (B - 1, B - 1)
    for (i, j) in upper:
        fin = i if j == B - 1 else -1
        if (i, j) in slot_of:
            step(prev[0], prev[1], B - 1, i, i, j, 0, 1, slot_of[(i, j)],
                 0, 0, fin)
        else:
            prev = (i, j)
            step(i, j, B - 1, i, i, j, 0, 0, -1, 0, 0, fin)
    return np.asarray(rows, dtype=np.int32)


def _body(sref, adj_ref, x_ref, w_ref, b_ref, out_ref,
          deg_ref, u_ref, s_ref, tc2_ref, mm_ref, cache_ref, *, T: int):
    t = pl.program_id(0)
    i = sref[t, _I]
    j = sref[t, _J]

    @pl.when(t == 0)
    def _init():
        deg_ref[...] = jnp.ones_like(deg_ref)
        u_ref[...] = jnp.zeros_like(u_ref)
        s_ref[...] = jnp.zeros_like(s_ref)
        tc2_ref[...] = jnp.ones_like(tc2_ref)

    tc2_ref[:, 0:1] = (jax.lax.rsqrt(deg_ref[j, :])
                       * s_ref[j, :]).reshape(T, 1)

    @pl.when(sref[t, _MV] != 1)
    def _mm_fetched():
        mm_ref[...] = jnp.dot(adj_ref[...], tc2_ref[...],
                              preferred_element_type=jnp.float32)

    @pl.when(sref[t, _MV] == 1)
    def _mm_cached():
        mm_ref[...] = jnp.dot(
            cache_ref[sref[t, _SLOT]],
            tc2_ref[...].astype(jnp.bfloat16),
            preferred_element_type=jnp.float32)

    rs_f = (sref[t, _RS] == 1).astype(jnp.float32)
    mv_f = (sref[t, _MV] >= 0).astype(jnp.float32)
    deg_ref[i, :] += rs_f * mm_ref[:, 1]
    u_ref[i, :] += mv_f * mm_ref[:, 0]

    @pl.when(sref[t, _STORE] == 1)
    def _park():
        cache_ref[sref[t, _SLOT]] = adj_ref[...].astype(jnp.bfloat16)

    @pl.when(sref[t, _DOS] == 1)
    def _diag():
        s_ref[i, :] = jnp.dot(
            x_ref[...], w_ref[...],
            preferred_element_type=jnp.float32).reshape(T)
        tcol = (jax.lax.rsqrt(deg_ref[i, :]) * s_ref[i, :]).reshape(T, 1)
        u_ref[i, :] += jnp.dot(
            adj_ref[...], tcol, preferred_element_type=jnp.float32)[:, 0]

    @pl.when(sref[t, _FIN] >= 0)
    def _finalize():
        r = sref[t, _FIN]
        d = jax.lax.rsqrt(deg_ref[r, :])
        tt = d * s_ref[r, :]
        out_ref[...] = (d * (u_ref[r, :] + tt) + b_ref[0, 0]).reshape(T, 1)


@functools.partial(jax.jit, static_argnames=("tile", "cache_tiles"))
def _run(x, adj, W, b2d, tile=1024, cache_tiles=21):
    n, f_in = x.shape
    T = tile
    B = n // T
    sched = _build_schedule(B, cache_tiles)
    K = max(1, min(cache_tiles, B * (B - 1) // 2))

    grid_spec = pltpu.PrefetchScalarGridSpec(
        num_scalar_prefetch=1,
        grid=(sched.shape[0],),
        in_specs=[
            pl.BlockSpec((T, T), lambda t, s: (s[t, _FI], s[t, _FJ])),
            pl.BlockSpec((T, f_in), lambda t, s: (s[t, _XI], 0)),
            pl.BlockSpec((f_in, 1), lambda t, s: (0, 0)),
            pl.BlockSpec((1, 1), lambda t, s: (0, 0)),
        ],
        out_specs=pl.BlockSpec((T, 1), lambda t, s: (s[t, _OI], 0)),
        scratch_shapes=[
            pltpu.VMEM((B, T), jnp.float32),
            pltpu.VMEM((B, T), jnp.float32),
            pltpu.VMEM((B, T), jnp.float32),
            pltpu.VMEM((T, 2), jnp.float32),
            pltpu.VMEM((T, 2), jnp.float32),
            pltpu.VMEM((K, T, T), jnp.bfloat16),
        ],
    )
    out = pl.pallas_call(
        functools.partial(_body, T=T),
        grid_spec=grid_spec,
        out_shape=jax.ShapeDtypeStruct((n, 1), jnp.float32),
    )(jnp.asarray(sched), adj, x, W, b2d)
    return out.reshape(1, -1)


def kernel(x, adj, W, b):
    return _run(x, adj, W, b.reshape(1, 1))


# speedup vs baseline: 1.2892x; 1.2892x over previous
import functools

import jax
import jax.numpy as jnp
import numpy as np
from jax.experimental import pallas as pl
from jax.experimental.pallas import tpu as pltpu

_FI, _FJ, _XI, _OI, _I, _J, _RS, _MV, _SLOT, _STORE, _DOS, _FIN = range(12)


def _build_schedule(B: int, K: int) -> np.ndarray:
    upper = [(i, j) for i in range(B) for j in range(i + 1, B)]
    K = min(K, len(upper))
    cached = upper[len(upper) - K:] if K else []
    slot_of = {t: s for s, t in enumerate(cached)}
    rows = []

    def step(fi, fj, xi, oi, i, j, rs, mv, slot, store, dos, fin):
        rows.append([fi, fj, xi, oi, i, j, rs, mv, slot, store, dos, fin])

    for i in range(B):
        order = list(range(i)) + list(range(i + 1, B)) + [i]
        for j in order:
            diag = j == i
            slot = slot_of.get((i, j), -1)
            step(i, j, i, B - 1, i, j, 1,
                 0 if (j < i) else -1, slot, 1 if slot >= 0 else 0,
                 1 if diag else 0,
                 B - 1 if (i == B - 1 and diag) else -1)
    prev = (B - 1, B - 1)
    for (i, j) in upper:
        fin = i if j == B - 1 else -1
        if (i, j) in slot_of:
            step(prev[0], prev[1], B - 1, i, i, j, 0, 1, slot_of[(i, j)],
                 0, 0, fin)
        else:
            prev = (i, j)
            step(i, j, B - 1, i, i, j, 0, 0, -1, 0, 0, fin)
    return np.asarray(rows, dtype=np.int32)


def _body(sref, adj_ref, x_ref, w_ref, b_ref, out_ref,
          deg_ref, u_ref, s_ref, tc2_ref, mm_ref, cache_ref, *, T: int):
    t = pl.program_id(0)
    i = sref[t, _I]
    j = sref[t, _J]

    @pl.when(t == 0)
    def _init():
        deg_ref[...] = jnp.ones_like(deg_ref)
        u_ref[...] = jnp.zeros_like(u_ref)
        s_ref[...] = jnp.zeros_like(s_ref)
        tc2_ref[...] = jnp.ones_like(tc2_ref)

    tc2_ref[:, 0:1] = (jax.lax.rsqrt(deg_ref[j, :])
                       * s_ref[j, :]).reshape(T, 1)

    @pl.when(sref[t, _MV] != 1)
    def _mm_fetched():
        mm_ref[...] = jnp.dot(adj_ref[...], tc2_ref[...],
                              preferred_element_type=jnp.float32).T

    @pl.when(sref[t, _MV] == 1)
    def _mm_cached():
        mm_ref[...] = jnp.dot(
            cache_ref[sref[t, _SLOT]],
            tc2_ref[...].astype(jnp.bfloat16),
            preferred_element_type=jnp.float32).T

    rs_f = (sref[t, _RS] == 1).astype(jnp.float32)
    mv_f = (sref[t, _MV] >= 0).astype(jnp.float32)
    deg_ref[i, :] += rs_f * mm_ref[1, :]
    u_ref[i, :] += mv_f * mm_ref[0, :]

    @pl.when(sref[t, _STORE] == 1)
    def _park():
        cache_ref[sref[t, _SLOT]] = adj_ref[...].astype(jnp.bfloat16)

    @pl.when(sref[t, _DOS] == 1)
    def _diag():
        s_ref[i, :] = jnp.dot(
            x_ref[...], w_ref[...],
            preferred_element_type=jnp.float32).reshape(T)
        tcol = (jax.lax.rsqrt(deg_ref[i, :]) * s_ref[i, :]).reshape(T, 1)
        u_ref[i, :] += jnp.dot(
            adj_ref[...], tcol, preferred_element_type=jnp.float32).T[0, :]

    @pl.when(sref[t, _FIN] >= 0)
    def _finalize():
        r = sref[t, _FIN]
        d = jax.lax.rsqrt(deg_ref[r, :])
        tt = d * s_ref[r, :]
        out_ref[...] = (d * (u_ref[r, :] + tt) + b_ref[0, 0]).reshape(T, 1)


@functools.partial(jax.jit, static_argnames=("tile", "cache_tiles"))
def _run(x, adj, W, b2d, tile=1024, cache_tiles=21):
    n, f_in = x.shape
    T = tile
    B = n // T
    sched = _build_schedule(B, cache_tiles)
    K = max(1, min(cache_tiles, B * (B - 1) // 2))

    grid_spec = pltpu.PrefetchScalarGridSpec(
        num_scalar_prefetch=1,
        grid=(sched.shape[0],),
        in_specs=[
            pl.BlockSpec((T, T), lambda t, s: (s[t, _FI], s[t, _FJ])),
            pl.BlockSpec((T, f_in), lambda t, s: (s[t, _XI], 0)),
            pl.BlockSpec((f_in, 1), lambda t, s: (0, 0)),
            pl.BlockSpec((1, 1), lambda t, s: (0, 0)),
        ],
        out_specs=pl.BlockSpec((T, 1), lambda t, s: (s[t, _OI], 0)),
        scratch_shapes=[
            pltpu.VMEM((B, T), jnp.float32),
            pltpu.VMEM((B, T), jnp.float32),
            pltpu.VMEM((B, T), jnp.float32),
            pltpu.VMEM((T, 2), jnp.float32),
            pltpu.VMEM((2, T), jnp.float32),
            pltpu.VMEM((K, T, T), jnp.bfloat16),
        ],
    )
    out = pl.pallas_call(
        functools.partial(_body, T=T),
        grid_spec=grid_spec,
        out_shape=jax.ShapeDtypeStruct((n, 1), jnp.float32),
    )(jnp.asarray(sched), adj, x, W, b2d)
    return out.reshape(1, -1)


def kernel(x, adj, W, b):
    return _run(x, adj, W, b.reshape(1, 1))
